# baseline (speedup 1.0000x reference)
import functools

import jax
import jax.numpy as jnp
from jax import lax
from jax.experimental import pallas as pl
from jax.experimental.pallas import tpu as pltpu
from jax.experimental.pallas import tpu_sc as plsc

B, C, T = 16, 9, 4096
L = 16
SEG = T // L
RW = L + 1
ZSZ = RW * SEG
OPAD = 3 * RW
OSZ = ZSZ + OPAD

_mesh = plsc.VectorSubcoreMesh(core_axis_name="c", subcore_axis_name="s")


@functools.partial(
    pl.kernel,
    mesh=_mesh,
    out_type=jax.ShapeDtypeStruct((B * 2 * T,), jnp.float32),
    compiler_params=pltpu.CompilerParams(needs_layout_passes=False),
    scratch_types=[
        pltpu.VMEM((T,), jnp.float32),
        pltpu.VMEM((T,), jnp.float32),
        pltpu.VMEM((ZSZ,), jnp.float32),
        pltpu.VMEM((ZSZ,), jnp.float32),
        pltpu.VMEM((ZSZ,), jnp.float32),
        pltpu.VMEM((OSZ,), jnp.float32),
        pltpu.VMEM((T,), jnp.float32),
        pltpu.VMEM((2 * L,), jnp.int32),
        pltpu.SemaphoreType.DMA,
        pltpu.SemaphoreType.DMA,
    ],
)
def _finger_mask_sc(gl_hbm, out_hbm, pbuf, rbuf, zp, zr, zs, zo, obuf, cbuf,
                    sem_p, sem_r):
    sid = lax.axis_index("s")

    iota = lax.iota(jnp.int32, L)
    iota17 = iota * RW
    one_i = jnp.ones((L,), jnp.int32)
    zero_i = jnp.zeros((L,), jnp.int32)
    one_f = jnp.ones((L,), jnp.float32)
    zero_f = jnp.zeros((L,), jnp.float32)

    def prefill(v, _):
        gidx = iota17 + (v * (RW * L) + L)
        plsc.store_scatter(zp, [gidx], zero_f)
        plsc.store_scatter(zr, [gidx], zero_f)
        plsc.store_scatter(zs, [gidx], zero_f)
        return 0

    lax.fori_loop(0, L, prefill, 0, unroll=4)

    def repack(m):
        sidx = iota17 + ((m % L) * (RW * L) + m // L)
        plsc.store_scatter(zp, [sidx], pbuf[pl.ds(m * L, L)])
        plsc.store_scatter(zr, [sidx], rbuf[pl.ds(m * L, L)])

    def process_row(wid):
        base = (wid // 2) * (9 * T) + (wid % 2) * (2 * T)
        cp_p = pltpu.async_copy(gl_hbm.at[pl.ds(base, T)], pbuf, sem_p)
        cp_r = pltpu.async_copy(gl_hbm.at[pl.ds(base + T, T)], rbuf, sem_r)
        cp_p.wait()
        cp_r.wait()

        plsc.parallel_loop(0, SEG, unroll=4)(repack)

        xm0 = zp[pl.ds(RW * (SEG - 1) - 1, L)]
        ym0 = zr[pl.ds(RW * (SEG - 1) - 1, L)]

        def pass1(j, carry):
            xm, ym, h, k, kp, kr = carry
            x = zp[pl.ds(j * RW, L)]
            y = zr[pl.ds(j * RW, L)]
            p_on = (x - xm) > 0
            r_on = (y - ym) > 0
            h = jnp.where(p_on, one_i, jnp.where(r_on, zero_i, h))
            k = jnp.where(p_on | r_on, one_i, k)
            kp = jnp.where(p_on, one_i, kp)
            kr = jnp.where(r_on, one_i, kr)
            return x, y, h, k, kp, kr

        _, _, h, k, kp, kr = lax.fori_loop(
            0, SEG, pass1, (xm0, ym0, zero_i, zero_i, zero_i, zero_i), unroll=4)

        valid = (jnp.max(kp) > 0) & (jnp.max(kr) > 0)
        vf = jnp.where(valid, jnp.float32(1.0), jnp.float32(0.0))

        e_lane = jnp.where(k > 0, iota * 2 + h, -one_i)
        ec = plsc.cummax(e_lane)
        cbuf[pl.ds(0, L)] = -one_i
        cbuf[pl.ds(1, L)] = ec
        ecs = cbuf[pl.ds(0, L)]
        s_entry = jnp.where(ecs >= 0, (ecs % 2).astype(jnp.float32), zero_f)

        def pass2(j, carry):
            xm, ym, s, s1, m21, m22, m41, m42, m43, m44 = carry
            x = zp[pl.ds(j * RW, L)]
            y = zr[pl.ds(j * RW, L)]
            p_on = (x - xm) > 0
            r_on = (y - ym) > 0
            s = jnp.where(p_on, one_f, jnp.where(r_on, zero_f, s))
            zs[pl.ds(j * RW, L)] = s
            m2 = jnp.maximum(s, s1)
            m4 = jnp.maximum(m2, m22)
            m8 = jnp.maximum(m4, m44)
            zo[pl.ds(j * RW, L)] = m8 * vf
            return x, y, s, s, m2, m21, m4, m41, m42, m43

        plsc.parallel_loop(
            0, SEG, unroll=4,
            carry=(xm0, ym0, s_entry, zero_f,
                   zero_f, zero_f, zero_f, zero_f, zero_f, zero_f))(pass2)

        for d in (0, 1, 2, 3, SEG - 3, SEG - 2, SEG - 1):
            m = None
            for kk in range(8):
                delta = d - 4 + kk
                if delta < 0:
                    addr = RW * (SEG + delta) - 1
                elif delta < SEG:
                    addr = RW * delta
                else:
                    addr = RW * (delta - SEG) + 1
                v = zs[pl.ds(addr, L)]
                m = v if m is None else jnp.maximum(m, v)
            zo[pl.ds(RW * d + OPAD, L)] = m * vf

        def unrepack(m):
            gidx = iota17 + ((m % L) * (RW * L) + m // L + OPAD)
            obuf[pl.ds(m * L, L)] = plsc.load_gather(zo, [gidx])

        plsc.parallel_loop(0, SEG, unroll=4)(unrepack)
        pltpu.sync_copy(obuf, out_hbm.at[pl.ds(wid * T, T)])

    process_row(sid * 2 + lax.axis_index("c"))


def kernel(gesture_labels):
    out = _finger_mask_sc(gesture_labels.reshape(-1))
    return out.reshape(B, 2, T)


# speedup vs baseline: 1.1159x; 1.1159x over previous
import functools

import jax
import jax.numpy as jnp
from jax import lax
from jax.experimental import pallas as pl
from jax.experimental.pallas import tpu as pltpu
from jax.experimental.pallas import tpu_sc as plsc

B, C, T = 16, 9, 4096
L = 16
SEG = T // L
RW = L + 1
ZSZ = RW * SEG
OPAD = 3 * RW
OSZ = ZSZ + OPAD

_mesh = plsc.VectorSubcoreMesh(core_axis_name="c", subcore_axis_name="s")


@functools.partial(
    pl.kernel,
    mesh=_mesh,
    out_type=jax.ShapeDtypeStruct((B * 2 * T,), jnp.float32),
    compiler_params=pltpu.CompilerParams(needs_layout_passes=False),
    scratch_types=[
        pltpu.VMEM((T,), jnp.float32),
        pltpu.VMEM((T,), jnp.float32),
        pltpu.VMEM((ZSZ,), jnp.float32),
        pltpu.VMEM((ZSZ,), jnp.float32),
        pltpu.VMEM((ZSZ,), jnp.float32),
        pltpu.VMEM((OSZ,), jnp.float32),
        pltpu.VMEM((T,), jnp.float32),
        pltpu.VMEM((2 * L,), jnp.int32),
        pltpu.SemaphoreType.DMA,
        pltpu.SemaphoreType.DMA,
    ],
)
def _finger_mask_sc(gl_hbm, out_hbm, pbuf, rbuf, zp, zr, zs, zo, obuf, cbuf,
                    sem_p, sem_r):
    sid = lax.axis_index("s")
    wid = sid * 2 + lax.axis_index("c")

    base = wid * (2 * T)
    cp_p = pltpu.async_copy(gl_hbm.at[pl.ds(base, T)], pbuf, sem_p)
    cp_r = pltpu.async_copy(gl_hbm.at[pl.ds(base + T, T)], rbuf, sem_r)

    iota = lax.iota(jnp.int32, L)
    iota17 = iota * RW
    one_i = jnp.ones((L,), jnp.int32)
    zero_i = jnp.zeros((L,), jnp.int32)
    one_f = jnp.ones((L,), jnp.float32)
    zero_f = jnp.zeros((L,), jnp.float32)

    def prefill(v, _):
        gidx = iota17 + (v * (RW * L) + L)
        plsc.store_scatter(zp, [gidx], zero_f)
        plsc.store_scatter(zr, [gidx], zero_f)
        plsc.store_scatter(zs, [gidx], zero_f)
        return 0

    lax.fori_loop(0, L, prefill, 0, unroll=4)

    def repack(m):
        sidx = iota17 + ((m % L) * (RW * L) + m // L)
        plsc.store_scatter(zp, [sidx], pbuf[pl.ds(m * L, L)])
        plsc.store_scatter(zr, [sidx], rbuf[pl.ds(m * L, L)])

    def process_row(wid):
        cp_p.wait()
        cp_r.wait()

        plsc.parallel_loop(0, SEG, unroll=4)(repack)

        xm0 = zp[pl.ds(RW * (SEG - 1) - 1, L)]
        ym0 = zr[pl.ds(RW * (SEG - 1) - 1, L)]

        def pass1(j, carry):
            xm, ym, h, k, kp, kr = carry
            x = zp[pl.ds(j * RW, L)]
            y = zr[pl.ds(j * RW, L)]
            p_on = (x - xm) > 0
            r_on = (y - ym) > 0
            h = jnp.where(p_on, one_i, jnp.where(r_on, zero_i, h))
            k = jnp.where(p_on | r_on, one_i, k)
            kp = jnp.where(p_on, one_i, kp)
            kr = jnp.where(r_on, one_i, kr)
            return x, y, h, k, kp, kr

        _, _, h, k, kp, kr = plsc.parallel_loop(
            0, SEG, unroll=4,
            carry=(xm0, ym0, zero_i, zero_i, zero_i, zero_i))(pass1)

        valid = (jnp.max(kp) > 0) & (jnp.max(kr) > 0)
        vf = jnp.where(valid, jnp.float32(1.0), jnp.float32(0.0))

        e_lane = jnp.where(k > 0, iota * 2 + h, -one_i)
        ec = plsc.cummax(e_lane)
        cbuf[pl.ds(0, L)] = -one_i
        cbuf[pl.ds(1, L)] = ec
        ecs = cbuf[pl.ds(0, L)]
        s_entry = jnp.where(ecs >= 0, (ecs % 2).astype(jnp.float32), zero_f)

        def pass2(j, carry):
            xm, ym, s, s1, m21, m22, m41, m42, m43, m44 = carry
            x = zp[pl.ds(j * RW, L)]
            y = zr[pl.ds(j * RW, L)]
            p_on = (x - xm) > 0
            r_on = (y - ym) > 0
            s = jnp.where(p_on, one_f, jnp.where(r_on, zero_f, s))
            zs[pl.ds(j * RW, L)] = s
            m2 = jnp.maximum(s, s1)
            m4 = jnp.maximum(m2, m22)
            m8 = jnp.maximum(m4, m44)
            zo[pl.ds(j * RW, L)] = m8 * vf
            return x, y, s, s, m2, m21, m4, m41, m42, m43

        plsc.parallel_loop(
            0, SEG, unroll=4,
            carry=(xm0, ym0, s_entry, zero_f,
                   zero_f, zero_f, zero_f, zero_f, zero_f, zero_f))(pass2)

        for d in (0, 1, 2, 3, SEG - 3, SEG - 2, SEG - 1):
            m = None
            for kk in range(8):
                delta = d - 4 + kk
                if delta < 0:
                    addr = RW * (SEG + delta) - 1
                elif delta < SEG:
                    addr = RW * delta
                else:
                    addr = RW * (delta - SEG) + 1
                v = zs[pl.ds(addr, L)]
                m = v if m is None else jnp.maximum(m, v)
            zo[pl.ds(RW * d + OPAD, L)] = m * vf

        def unrepack(m):
            gidx = iota17 + ((m % L) * (RW * L) + m // L + OPAD)
            obuf[pl.ds(m * L, L)] = plsc.load_gather(zo, [gidx])

        plsc.parallel_loop(0, SEG, unroll=4)(unrepack)
        pltpu.sync_copy(obuf, out_hbm.at[pl.ds(wid * T, T)])

    process_row(wid)


def kernel(gesture_labels):
    gl4 = gesture_labels[:, :4, :].reshape(-1)
    out = _finger_mask_sc(gl4)
    return out.reshape(B, 2, T)
